# Initial kernel scaffold; baseline (speedup 1.0000x reference)
import jax
import jax.numpy as jnp
from jax.experimental import pallas as pl


def kernel(x, pos, edge_index, W, b, gamma, beta):
    raise NotImplementedError("write your pallas kernel here")


# speedup vs baseline: 1.6126x; 1.6126x over previous
import jax
import jax.numpy as jnp
from jax.experimental import pallas as pl
from jax.experimental.pallas import tpu as pltpu

N_PAD_M = 10240
BM = 1024


def _matmul_body(feat_ref, w_ref, out_ref):
    out_ref[...] = jnp.dot(feat_ref[...], w_ref[...],
                           preferred_element_type=jnp.float32)


def _node_transform(feat_pad, w_cat):
    grid = (N_PAD_M // BM,)
    return pl.pallas_call(
        _matmul_body,
        grid=grid,
        in_specs=[
            pl.BlockSpec((BM, 384), lambda i: (i, 0)),
            pl.BlockSpec((384, 512), lambda i: (0, 0)),
        ],
        out_specs=pl.BlockSpec((BM, 512), lambda i: (i, 0)),
        out_shape=jax.ShapeDtypeStruct((N_PAD_M, 512), jnp.float32),
    )(feat_pad, w_cat)


def _epilogue_body(m_ref, pw_ref, b_ref, gamma_ref, beta_ref, out_ref):
    n_valid = 10000.0
    m = m_ref[...]
    agg = m - pw_ref[...] + b_ref[...]
    valid = m > -1e37
    h = jnp.where(valid, jnp.where(agg > 0, agg, jnp.exp(jnp.minimum(agg, 0.0)) - 1.0), 0.0)
    s1 = jnp.sum(h, axis=0, keepdims=True)
    s2 = jnp.sum(h * h, axis=0, keepdims=True)
    mean = s1 / n_valid
    var = s2 / n_valid - mean * mean
    inv = jax.lax.rsqrt(var + 1e-5)
    out_ref[...] = (h - mean) * inv * gamma_ref[...] + beta_ref[...]


def _epilogue(m, pw, b, gamma, beta):
    np_rows = m.shape[0]
    return pl.pallas_call(
        _epilogue_body,
        out_shape=jax.ShapeDtypeStruct((np_rows, 256), jnp.float32),
    )(m, pw, b.reshape(1, 256), gamma.reshape(1, 256), beta.reshape(1, 256))


def kernel(x, pos, edge_index, W, b, gamma, beta):
    n = x.shape[0]
    feat = jnp.concatenate([x, pos], axis=1)
    feat_pad = jnp.zeros((N_PAD_M, 384), jnp.float32).at[:n, :259].set(feat)
    w_u = jnp.zeros((384, 256), jnp.float32).at[:259, :].set(W)
    w_pw = jnp.zeros((384, 256), jnp.float32).at[256:259, :].set(W[256:259])
    w_cat = jnp.concatenate([w_u, w_pw], axis=1)
    uc = _node_transform(feat_pad, w_cat)
    u = uc[:n, :256]
    pw = uc[:, 256:512]

    src = edge_index[0]
    dst = edge_index[1]
    m = jax.ops.segment_max(u[src], dst, num_segments=N_PAD_M)

    out = _epilogue(m, pw, b, gamma, beta)
    return out[:n]
